# Initial kernel scaffold; baseline (speedup 1.0000x reference)
import jax
import jax.numpy as jnp
from jax.experimental import pallas as pl


def kernel(x, edge_index, edge_weights, W, bias, gamma, beta):
    raise NotImplementedError("write your pallas kernel here")


# speedup vs baseline: 1.1423x; 1.1423x over previous
import jax
import jax.numpy as jnp
from jax.experimental import pallas as pl
from jax.experimental.pallas import tpu as pltpu


def _bn_relu_body(pre_ref, gamma_ref, beta_ref, out_ref):
    v = pre_ref[...]
    n = v.shape[0]
    mean = jnp.sum(v, axis=0, keepdims=True) / n
    d = v - mean
    var = jnp.sum(d * d, axis=0, keepdims=True) / n
    inv = jax.lax.rsqrt(var + 1e-5)
    out_ref[...] = jnp.maximum(d * inv * gamma_ref[...] + beta_ref[...], 0.0)


def kernel(x, edge_index, edge_weights, W, bias, gamma, beta):
    n = x.shape[0]
    mask = edge_index[0] != edge_index[1]
    ew = jnp.where(mask, edge_weights, jnp.zeros((), dtype=edge_weights.dtype))
    row = edge_index[0]
    col = edge_index[1]
    deg = jnp.ones((n,), dtype=x.dtype).at[col].add(ew)
    dis = jax.lax.rsqrt(deg)
    norm = dis[row] * ew * dis[col]
    h = x @ W
    msg = h[row] * norm[:, None]
    out = (h * (1.0 / deg)[:, None]).at[col].add(msg)
    out = out + bias

    return pl.pallas_call(
        _bn_relu_body,
        out_shape=jax.ShapeDtypeStruct(out.shape, out.dtype),
    )(out, gamma.reshape(1, -1), beta.reshape(1, -1))


# speedup vs baseline: 11.7975x; 10.3282x over previous
import functools

import jax
import jax.numpy as jnp
from jax import lax
from jax.experimental import pallas as pl
from jax.experimental.pallas import tpu as pltpu
from jax.experimental.pallas import tpu_sc as plsc

NC = 2
NS = 16
L = 16
NW = NC * NS
C = 128
NP = 2


def _matmul_body(x_ref, w_ref, o_ref):
    o_ref[...] = jnp.dot(x_ref[...], w_ref[...],
                         preferred_element_type=jnp.float32)


def _bn_relu_body(p_ref, bias_ref, gamma_ref, beta_ref, o_ref):
    n = o_ref.shape[0]
    pre = jnp.concatenate(
        [p_ref[0, 0] + p_ref[1, 0], p_ref[0, 1] + p_ref[1, 1]], axis=1)
    pre = pre[:n] + bias_ref[...]
    mean = jnp.sum(pre, axis=0, keepdims=True) * (1.0 / n)
    d = pre - mean
    var = jnp.sum(d * d, axis=0, keepdims=True) * (1.0 / n)
    inv = lax.rsqrt(var + 1e-5)
    o_ref[...] = jnp.maximum(d * inv * gamma_ref[...] + beta_ref[...], 0.0)


def _rsqrt16(d):
    i = plsc.bitcast(d, jnp.int32)
    i = jnp.int32(0x5F3759DF) - (i >> 1)
    y = plsc.bitcast(i, jnp.float32)
    for _ in range(3):
        y = y * (1.5 - 0.5 * d * y * y)
    return y


def _make_sc_kernel(Npad, D, J):
    SEG = Npad // NS
    K = SEG // C
    DP = D // NP
    mesh = plsc.VectorSubcoreMesh(
        core_axis_name="c", subcore_axis_name="s",
        num_cores=NC, num_subcores=NS)

    @functools.partial(
        pl.kernel,
        out_type=jax.ShapeDtypeStruct((NC, NP, Npad, DP), jnp.float32),
        mesh=mesh,
        compiler_params=pltpu.CompilerParams(
            needs_layout_passes=False, use_tc_tiling_on_sc=False),
        scratch_types=[
            pltpu.VMEM((J, C), jnp.int32),
            pltpu.VMEM((J, C), jnp.int32),
            pltpu.VMEM((J, C), jnp.float32),
            pltpu.VMEM((Npad,), jnp.float32),
            pltpu.VMEM((Npad,), jnp.float32),
            pltpu.VMEM((J * C,), jnp.float32),
            pltpu.VMEM((C, DP), jnp.float32),
            pltpu.VMEM((SEG,), jnp.float32),
            pltpu.VMEM((SEG,), jnp.float32),
            pltpu.VMEM_SHARED((NS, Npad), jnp.float32),
            pltpu.VMEM_SHARED((Npad,), jnp.float32),
            pltpu.VMEM_SHARED((Npad, DP), jnp.float32),
            pltpu.SemaphoreType.DMA,
        ],
    )
    def sc_kernel(row_hbm, col_hbm, ew_hbm, h_hbm, outp_hbm,
                  rowbuf, colbuf, ewbuf, degv, disv, sbuf, rows_a,
                  tmpseg, accseg, deg_slab, dis_slab, out_acc, semg):
        cid = lax.axis_index("c")
        sid = lax.axis_index("s")
        wid = cid * NS + sid
        seg0 = sid * SEG
        zero16 = jnp.zeros((L,), jnp.float32)

        def zdeg(i, _):
            degv[pl.ds(i * L, L)] = zero16
            return _
        lax.fori_loop(0, Npad // L, zdeg, None)

        for b in range(2):
            blk = sid * 2 + b
            pltpu.sync_copy(row_hbm.at[blk], rowbuf)
            pltpu.sync_copy(col_hbm.at[blk], colbuf)
            pltpu.sync_copy(ew_hbm.at[blk], ewbuf)

            def degacc(j, _):
                for g in range(C // L):
                    sl = pl.ds(g * L, L)
                    r16 = rowbuf[j, sl]
                    c16 = colbuf[j, sl]
                    w16 = ewbuf[j, sl]
                    wm = jnp.where(r16 != c16, w16, 0.0)
                    plsc.addupdate_scatter(degv, [c16], wm)
                return _
            lax.fori_loop(0, J, degacc, None)

        pltpu.sync_copy(degv, deg_slab.at[sid])
        plsc.subcore_barrier()

        def zacc(i, _):
            accseg[pl.ds(i * L, L)] = zero16
            return _
        lax.fori_loop(0, SEG // L, zacc, None)
        for k in range(NS):
            pltpu.sync_copy(deg_slab.at[k, pl.ds(seg0, SEG)], tmpseg)

            def radd(i, _):
                sl = pl.ds(i * L, L)
                accseg[sl] = accseg[sl] + tmpseg[sl]
                return _
            lax.fori_loop(0, SEG // L, radd, None)

        def rsq(i, _):
            sl = pl.ds(i * L, L)
            d = accseg[sl] + 1.0
            accseg[sl] = _rsqrt16(d)
            return _
        lax.fori_loop(0, SEG // L, rsq, None)

        pltpu.sync_copy(accseg, dis_slab.at[pl.ds(seg0, SEG)])
        plsc.subcore_barrier()
        pltpu.sync_copy(dis_slab, disv)

        pltpu.sync_copy(row_hbm.at[wid], rowbuf)
        pltpu.sync_copy(col_hbm.at[wid], colbuf)
        pltpu.sync_copy(ew_hbm.at[wid], ewbuf)

        cfac = jnp.where(cid == 0, 1.0, 0.0).astype(jnp.float32)

        for p in range(NP):
            for k in range(K):
                base = seg0 + k * C
                pltpu.sync_copy(h_hbm.at[p, pl.ds(base, C)], rows_a)

                def selfloop(e, _):
                    idx16 = jnp.full((L,), base + e, jnp.int32)
                    dv = plsc.load_gather(disv, [idx16])
                    sq = dv * dv * cfac
                    for d in range(DP // L):
                        sl = pl.ds(d * L, L)
                        rows_a[e, sl] = rows_a[e, sl] * sq
                    return _
                lax.fori_loop(0, C, selfloop, None)
                pltpu.sync_copy(rows_a, out_acc.at[pl.ds(base, C)])
            plsc.subcore_barrier()

            def chunk(j, _):
                cp = pltpu.async_copy(
                    h_hbm.at[p].at[rowbuf.at[j]], rows_a, semg)
                if p == 0:
                    for g in range(C // L):
                        sl = pl.ds(g * L, L)
                        r16 = rowbuf[j, sl]
                        c16 = colbuf[j, sl]
                        w16 = ewbuf[j, sl]
                        wm = jnp.where(r16 != c16, w16, 0.0)
                        s = plsc.load_gather(disv, [r16]) * wm \
                            * plsc.load_gather(disv, [c16])
                        sbuf[pl.ds(j * C + g * L, L)] = s
                cp.wait()

                def scale(e, _):
                    sv = plsc.load_gather(
                        sbuf, [jnp.full((L,), j * C + e, jnp.int32)])
                    for d in range(DP // L):
                        sl = pl.ds(d * L, L)
                        rows_a[e, sl] = rows_a[e, sl] * sv
                    return _
                lax.fori_loop(0, C, scale, None)
                pltpu.sync_copy(rows_a, out_acc.at[colbuf.at[j]], add=True)
                return _
            lax.fori_loop(0, J, chunk, None)

            plsc.subcore_barrier()
            pltpu.sync_copy(out_acc.at[pl.ds(seg0, SEG)],
                            outp_hbm.at[cid, p, pl.ds(seg0, SEG)])
            if p + 1 < NP:
                plsc.subcore_barrier()

    return sc_kernel


def kernel(x, edge_index, edge_weights, W, bias, gamma, beta):
    n, d_in = x.shape
    d_out = W.shape[1]
    e = edge_weights.shape[0]

    npad = ((n + NS * C - 1) // (NS * C)) * (NS * C)
    j_chunks = (e + NW * C - 1) // (NW * C)
    e_pad = NW * j_chunks * C
    dp = d_out // NP

    row3 = jnp.pad(edge_index[0], (0, e_pad - e)).reshape(NW, j_chunks, C)
    col3 = jnp.pad(edge_index[1], (0, e_pad - e)).reshape(NW, j_chunks, C)
    ew3 = jnp.pad(edge_weights, (0, e_pad - e)).reshape(NW, j_chunks, C)
    x_p = jnp.pad(x, ((0, npad - n), (0, 0)))

    blk = npad // 8
    h = pl.pallas_call(
        _matmul_body,
        grid=(8,),
        in_specs=[
            pl.BlockSpec((blk, d_in), lambda i: (i, 0)),
            pl.BlockSpec((d_in, d_out), lambda i: (0, 0)),
        ],
        out_specs=pl.BlockSpec((blk, d_out), lambda i: (i, 0)),
        out_shape=jax.ShapeDtypeStruct((npad, d_out), jnp.float32),
    )(x_p, W)

    h2 = jnp.stack([h[:, :dp], h[:, dp:]])

    outp = _make_sc_kernel(npad, d_out, j_chunks)(row3, col3, ew3, h2)

    return pl.pallas_call(
        _bn_relu_body,
        out_shape=jax.ShapeDtypeStruct((n, d_out), jnp.float32),
    )(outp, bias.reshape(1, -1), gamma.reshape(1, -1), beta.reshape(1, -1))
